# baseline (speedup 1.0000x reference)
import jax
import jax.numpy as jnp
from jax.experimental import pallas as pl
from jax.experimental.pallas import tpu as pltpu

_BR = 512


def _softmax_threshold(s, out_ref):
    m = jnp.max(s, axis=-1, keepdims=True)
    e = jnp.exp2(s - m)
    ssum = jnp.sum(e, axis=-1, keepdims=True)
    thresh = ssum * (1.0 / s.shape[-1])
    out_ref[...] = jnp.where(e < thresh, 0.0, e) * (1.0 / ssum)


def _sim_kernel(rows_ref, nodes_ref, out_ref, s0_ref, s1_ref):
    i = pl.program_id(0)
    a = rows_ref[...] * jnp.float32(1.4426950408889634)
    b = nodes_ref[...]

    def _scores():
        return jax.lax.dot_general(
            a, b, (((1,), (1,)), ((), ())),
            preferred_element_type=jnp.float32)

    @pl.when(jax.lax.rem(i, 2) == 0)
    def _():
        s0_ref[...] = _scores()
        _softmax_threshold(s1_ref[...], out_ref)

    @pl.when(jax.lax.rem(i, 2) == 1)
    def _():
        s1_ref[...] = _scores()
        _softmax_threshold(s0_ref[...], out_ref)


@jax.jit
def kernel(X):
    H, W, C = X.shape
    n = H * W
    nodes = X.reshape(n, C)
    nb = n // _BR
    return pl.pallas_call(
        _sim_kernel,
        grid=(nb + 1,),
        in_specs=[
            pl.BlockSpec((_BR, C), lambda i: (jnp.minimum(i, nb - 1), 0)),
            pl.BlockSpec((n, C), lambda i: (0, 0)),
        ],
        out_specs=pl.BlockSpec((_BR, n), lambda i: (jnp.maximum(i - 1, 0), 0)),
        out_shape=jax.ShapeDtypeStruct((n, n), jnp.float32),
        scratch_shapes=[
            pltpu.VMEM((_BR, n), jnp.float32),
            pltpu.VMEM((_BR, n), jnp.float32),
        ],
    )(nodes, nodes)


# speedup vs baseline: 1.2099x; 1.2099x over previous
import jax
import jax.numpy as jnp
from jax.experimental import pallas as pl

_BR = 512


def _sim_kernel(rows_ref, nodes_ref, out_ref):
    a = rows_ref[...] * jnp.float32(1.4426950408889634)
    b = nodes_ref[...]
    s = jax.lax.dot_general(
        a, b, (((1,), (1,)), ((), ())),
        preferred_element_type=jnp.float32)
    m = jnp.max(s, axis=-1, keepdims=True)
    e = jnp.exp2(s - m)
    ssum = jnp.sum(e, axis=-1, keepdims=True)
    thresh = ssum * (1.0 / s.shape[-1])
    out_ref[...] = jnp.where(e < thresh, 0.0, e) * (1.0 / ssum)


@jax.jit
def kernel(X):
    H, W, C = X.shape
    n = H * W
    nodes = X.reshape(n, C)
    grid = (n // _BR,)
    return pl.pallas_call(
        _sim_kernel,
        grid=grid,
        in_specs=[
            pl.BlockSpec((_BR, C), lambda i: (i, 0)),
            pl.BlockSpec((n, C), lambda i: (0, 0)),
        ],
        out_specs=pl.BlockSpec((_BR, n), lambda i: (i, 0)),
        out_shape=jax.ShapeDtypeStruct((n, n), jnp.float32),
    )(nodes, nodes)
